# baseline (speedup 1.0000x reference)
import functools

import jax
import jax.numpy as jnp
from jax import lax
from jax.experimental import pallas as pl
from jax.experimental.pallas import tpu as pltpu
from jax.experimental.pallas import tpu_sc as plsc

_N = 10000
_D = 128
_E = 320000
_NC = 2
_NS = 16
_NW = _NC * _NS
_EW = _E // _NW
_CH = 128
_K = 80
_NPH = 2
_KP = _K // _NPH
_KP2 = _KP // 2
_PAD = _K * _CH - _EW
_NDUM = 112
_NACC = _N + _NDUM
_RZ = _NACC // _NS

_MESH = dict(core_axis_name="c", subcore_axis_name="s")


@functools.partial(
    pl.kernel,
    mesh=plsc.VectorSubcoreMesh(**_MESH),
    out_type=jax.ShapeDtypeStruct((_NC, _NACC, _D), jnp.float32),
    scratch_types=[
        pltpu.VMEM((_K, _CH), jnp.int32),
        pltpu.VMEM((_CH, _D), jnp.float32),
        pltpu.VMEM_SHARED((_NACC, _D), jnp.float32),
        pltpu.SemaphoreType.DMA,
    ],
)
def _deg_kernel(dst_hbm, z_hbm, ones_hbm, out_hbm, dst_v, ones_v, acc, sem):
    cid = lax.axis_index("c")
    sid = lax.axis_index("s")
    w = cid * _NS + sid
    pltpu.sync_copy(z_hbm, acc.at[pl.ds(sid * _RZ, _RZ)])
    pltpu.sync_copy(dst_hbm.at[w], dst_v)
    pltpu.sync_copy(ones_hbm, ones_v)
    plsc.subcore_barrier()

    def fire(j, carry):
        pltpu.async_copy(ones_v, acc.at[dst_v.at[j]], sem, add=True)
        return carry

    def drain(j, carry):
        pltpu.make_async_copy(ones_v, acc.at[dst_v.at[j]], sem).wait()
        return carry

    lax.fori_loop(0, _K, fire, 0)
    lax.fori_loop(0, _K, drain, 0)
    plsc.subcore_barrier()
    pltpu.sync_copy(acc.at[pl.ds(sid * _RZ, _RZ)],
                    out_hbm.at[cid, pl.ds(sid * _RZ, _RZ)])


@functools.partial(
    pl.kernel,
    mesh=plsc.VectorSubcoreMesh(**_MESH),
    out_type=jax.ShapeDtypeStruct((_NC, _NACC, _D), jnp.float32),
    scratch_types=[
        pltpu.VMEM((_KP, _CH), jnp.int32),
        pltpu.VMEM((_KP, _CH), jnp.int32),
        pltpu.VMEM((_CH, _D), jnp.float32),
        pltpu.VMEM((_CH, _D), jnp.float32),
        pltpu.VMEM_SHARED((_NACC, _D), jnp.float32),
        pltpu.SemaphoreType.DMA,
        pltpu.SemaphoreType.DMA,
        pltpu.SemaphoreType.DMA,
        pltpu.SemaphoreType.DMA,
    ],
)
def _segsum(h_hbm, src_hbm, dst_hbm, z_hbm, out_hbm,
            src_v, dst_v, rows0, rows1, acc, sem0, sem1, semS0, semS1):
    cid = lax.axis_index("c")
    sid = lax.axis_index("s")
    w = cid * _NS + sid
    pltpu.sync_copy(z_hbm, acc.at[pl.ds(sid * _RZ, _RZ)])
    plsc.subcore_barrier()

    def pair(i, carry):
        a = 2 * i
        b = a + 1
        pltpu.make_async_copy(h_hbm.at[src_v.at[a]], rows0, sem0).wait()
        pltpu.async_copy(rows0, acc.at[dst_v.at[a]], semS0, add=True)
        pltpu.make_async_copy(h_hbm.at[src_v.at[b]], rows1, sem1).wait()
        pltpu.async_copy(rows1, acc.at[dst_v.at[b]], semS1, add=True)
        pltpu.make_async_copy(rows0, acc.at[dst_v.at[a]], semS0).wait()

        @pl.when(i < _KP2 - 1)
        def _():
            pltpu.async_copy(h_hbm.at[src_v.at[a + 2]], rows0, sem0)

        pltpu.make_async_copy(rows1, acc.at[dst_v.at[b]], semS1).wait()

        @pl.when(i < _KP2 - 1)
        def _():
            pltpu.async_copy(h_hbm.at[src_v.at[b + 2]], rows1, sem1)

        return carry

    for p in range(_NPH):
        pltpu.sync_copy(src_hbm.at[w, pl.ds(p * _KP, _KP)], src_v)
        pltpu.sync_copy(dst_hbm.at[w, pl.ds(p * _KP, _KP)], dst_v)
        pltpu.async_copy(h_hbm.at[src_v.at[0]], rows0, sem0)
        pltpu.async_copy(h_hbm.at[src_v.at[1]], rows1, sem1)
        lax.fori_loop(0, _KP2, pair, 0)
    plsc.subcore_barrier()
    pltpu.sync_copy(acc.at[pl.ds(sid * _RZ, _RZ)],
                    out_hbm.at[cid, pl.ds(sid * _RZ, _RZ)])


_B = 400
_G = _N // _B


def _dis_block(degp):
    deg = degp[0, :, 0:1] + degp[1, :, 0:1]
    return lax.rsqrt(jnp.maximum(deg, 1.0))


def _tc_lin1_body(x_ref, w_ref, degp_ref, o_ref):
    dis = _dis_block(degp_ref[...])
    o_ref[...] = jnp.dot(x_ref[...], w_ref[...],
                         preferred_element_type=jnp.float32) * dis


def _tc_lin2_body(p_ref, degp_ref, b1_ref, w_ref, o_ref):
    dis = _dis_block(degp_ref[...])
    p = p_ref[...]
    h = jnp.maximum((p[0] + p[1]) * dis + b1_ref[...], 0.0)
    o_ref[...] = jnp.dot(h, w_ref[...],
                         preferred_element_type=jnp.float32) * dis


def _tc_out_body(q_ref, degp_ref, b2_ref, o_ref):
    dis = _dis_block(degp_ref[...])
    q = q_ref[...]
    o_ref[...] = (q[0] + q[1]) * dis + b2_ref[...]


def _tc_lin1(x, W1, degp):
    return pl.pallas_call(
        _tc_lin1_body,
        grid=(_G,),
        in_specs=[
            pl.BlockSpec((_B, _D), lambda i: (i, 0)),
            pl.BlockSpec((_D, _D), lambda i: (0, 0)),
            pl.BlockSpec((_NC, _B, _D), lambda i: (0, i, 0)),
        ],
        out_specs=pl.BlockSpec((_B, _D), lambda i: (i, 0)),
        out_shape=jax.ShapeDtypeStruct((_N, _D), jnp.float32),
    )(x, W1, degp)


def _tc_lin2(p, degp, b1, W2):
    return pl.pallas_call(
        _tc_lin2_body,
        grid=(_G,),
        in_specs=[
            pl.BlockSpec((_NC, _B, _D), lambda i: (0, i, 0)),
            pl.BlockSpec((_NC, _B, _D), lambda i: (0, i, 0)),
            pl.BlockSpec((1, _D), lambda i: (0, 0)),
            pl.BlockSpec((_D, _D), lambda i: (0, 0)),
        ],
        out_specs=pl.BlockSpec((_B, _D), lambda i: (i, 0)),
        out_shape=jax.ShapeDtypeStruct((_N, _D), jnp.float32),
    )(p, degp, b1, W2)


def _tc_out(q, degp, b2):
    return pl.pallas_call(
        _tc_out_body,
        grid=(_G,),
        in_specs=[
            pl.BlockSpec((_NC, _B, _D), lambda i: (0, i, 0)),
            pl.BlockSpec((_NC, _B, _D), lambda i: (0, i, 0)),
            pl.BlockSpec((1, _D), lambda i: (0, 0)),
        ],
        out_specs=pl.BlockSpec((_B, _D), lambda i: (i, 0)),
        out_shape=jax.ShapeDtypeStruct((_N, _D), jnp.float32),
    )(q, degp, b2)


def kernel(x, edge_index, W1, b1, W2, b2):
    src = edge_index[0]
    dst = edge_index[1]
    pad_src = jnp.broadcast_to(
        jnp.arange(_PAD, dtype=jnp.int32) % 16, (_NW, _PAD))
    pad_dst = jnp.broadcast_to(
        jnp.arange(_PAD, dtype=jnp.int32) % _NDUM + _N, (_NW, _PAD))
    srcw = jnp.concatenate(
        [src.reshape(_NW, _EW), pad_src], axis=1).reshape(_NW, _K, _CH)
    dstw = jnp.concatenate(
        [dst.reshape(_NW, _EW), pad_dst], axis=1).reshape(_NW, _K, _CH)
    z128 = jnp.zeros((_RZ, _D), jnp.float32)
    ones128 = jnp.ones((_CH, _D), jnp.float32)

    degp = _deg_kernel(dstw, z128, ones128)
    h1 = _tc_lin1(x, W1, degp)
    p1 = _segsum(h1, srcw, dstw, z128)
    h2 = _tc_lin2(p1, degp, b1.reshape(1, _D), W2)
    p2 = _segsum(h2, srcw, dstw, z128)
    return _tc_out(p2, degp, b2.reshape(1, _D))


# speedup vs baseline: 1.1629x; 1.1629x over previous
import functools

import jax
import jax.numpy as jnp
from jax import lax
from jax.experimental import pallas as pl
from jax.experimental.pallas import tpu as pltpu
from jax.experimental.pallas import tpu_sc as plsc

_N = 10000
_D = 128
_E = 320000
_NC = 2
_NS = 16
_NW = _NC * _NS
_EW = _E // _NW
_CH = 128
_K = 80
_NPH = 2
_KP = _K // _NPH
_KP2 = _KP // 2
_PAD = _K * _CH - _EW
_NDUM = 112
_NACC = _N + _NDUM
_RZ = _NACC // _NS

_MESH = dict(core_axis_name="c", subcore_axis_name="s")


@functools.partial(
    pl.kernel,
    mesh=plsc.VectorSubcoreMesh(**_MESH),
    out_type=jax.ShapeDtypeStruct((_NC, _NACC, _D), jnp.float32),
    scratch_types=[
        pltpu.VMEM((_K, _CH), jnp.int32),
        pltpu.VMEM((_CH, _D), jnp.float32),
        pltpu.VMEM_SHARED((_NACC, _D), jnp.float32),
        pltpu.SemaphoreType.DMA,
    ],
)
def _deg_kernel(dst_hbm, z_hbm, ones_hbm, out_hbm, dst_v, ones_v, acc, sem):
    cid = lax.axis_index("c")
    sid = lax.axis_index("s")
    w = cid * _NS + sid
    pltpu.sync_copy(z_hbm, acc.at[pl.ds(sid * _RZ, _RZ)])
    pltpu.sync_copy(dst_hbm.at[w], dst_v)
    pltpu.sync_copy(ones_hbm, ones_v)
    plsc.subcore_barrier()

    def fire(j, carry):
        pltpu.async_copy(ones_v, acc.at[dst_v.at[j]], sem, add=True)
        return carry

    def drain(j, carry):
        pltpu.make_async_copy(ones_v, acc.at[dst_v.at[j]], sem).wait()
        return carry

    lax.fori_loop(0, _K, fire, 0)
    lax.fori_loop(0, _K, drain, 0)
    plsc.subcore_barrier()
    pltpu.sync_copy(acc.at[pl.ds(sid * _RZ, _RZ)],
                    out_hbm.at[cid, pl.ds(sid * _RZ, _RZ)])


@functools.partial(
    pl.kernel,
    mesh=plsc.VectorSubcoreMesh(**_MESH),
    out_type=jax.ShapeDtypeStruct((_NC, _NACC, _D), jnp.float32),
    scratch_types=[
        pltpu.VMEM((_KP, _CH), jnp.int32),
        pltpu.VMEM((_KP, _CH), jnp.int32),
        pltpu.VMEM((_CH, _D), jnp.float32),
        pltpu.VMEM((_CH, _D), jnp.float32),
        pltpu.VMEM_SHARED((_NACC, _D), jnp.float32),
        pltpu.SemaphoreType.DMA,
        pltpu.SemaphoreType.DMA,
    ],
)
def _segsum(h_hbm, src_hbm, dst_hbm, z_hbm, out_hbm,
            src_v, dst_v, rows0, rows1, acc, sem0, sem1):
    cid = lax.axis_index("c")
    sid = lax.axis_index("s")
    w = cid * _NS + sid
    pltpu.sync_copy(z_hbm, acc.at[pl.ds(sid * _RZ, _RZ)])
    plsc.subcore_barrier()

    def pair(i, carry):
        a = 2 * i
        b = a + 1
        pltpu.async_copy(h_hbm.at[src_v.at[b]], rows1, sem1)
        pltpu.make_async_copy(h_hbm.at[src_v.at[a]], rows0, sem0).wait()
        pltpu.sync_copy(rows0, acc.at[dst_v.at[a]], add=True)

        @pl.when(i < _KP2 - 1)
        def _():
            pltpu.async_copy(h_hbm.at[src_v.at[a + 2]], rows0, sem0)

        pltpu.make_async_copy(h_hbm.at[src_v.at[b]], rows1, sem1).wait()
        pltpu.sync_copy(rows1, acc.at[dst_v.at[b]], add=True)
        return carry

    for p in range(_NPH):
        pltpu.sync_copy(src_hbm.at[w, pl.ds(p * _KP, _KP)], src_v)
        pltpu.sync_copy(dst_hbm.at[w, pl.ds(p * _KP, _KP)], dst_v)
        pltpu.async_copy(h_hbm.at[src_v.at[0]], rows0, sem0)
        lax.fori_loop(0, _KP2, pair, 0)
    plsc.subcore_barrier()
    pltpu.sync_copy(acc.at[pl.ds(sid * _RZ, _RZ)],
                    out_hbm.at[cid, pl.ds(sid * _RZ, _RZ)])


_B = 400
_G = _N // _B


def _dis_block(degp):
    deg = degp[0, :, 0:1] + degp[1, :, 0:1]
    return lax.rsqrt(jnp.maximum(deg, 1.0))


def _tc_lin1_body(x_ref, w_ref, degp_ref, o_ref):
    dis = _dis_block(degp_ref[...])
    o_ref[...] = jnp.dot(x_ref[...], w_ref[...],
                         preferred_element_type=jnp.float32) * dis


def _tc_lin2_body(p_ref, degp_ref, b1_ref, w_ref, o_ref):
    dis = _dis_block(degp_ref[...])
    p = p_ref[...]
    h = jnp.maximum((p[0] + p[1]) * dis + b1_ref[...], 0.0)
    o_ref[...] = jnp.dot(h, w_ref[...],
                         preferred_element_type=jnp.float32) * dis


def _tc_out_body(q_ref, degp_ref, b2_ref, o_ref):
    dis = _dis_block(degp_ref[...])
    q = q_ref[...]
    o_ref[...] = (q[0] + q[1]) * dis + b2_ref[...]


def _tc_lin1(x, W1, degp):
    return pl.pallas_call(
        _tc_lin1_body,
        grid=(_G,),
        in_specs=[
            pl.BlockSpec((_B, _D), lambda i: (i, 0)),
            pl.BlockSpec((_D, _D), lambda i: (0, 0)),
            pl.BlockSpec((_NC, _B, _D), lambda i: (0, i, 0)),
        ],
        out_specs=pl.BlockSpec((_B, _D), lambda i: (i, 0)),
        out_shape=jax.ShapeDtypeStruct((_N, _D), jnp.float32),
    )(x, W1, degp)


def _tc_lin2(p, degp, b1, W2):
    return pl.pallas_call(
        _tc_lin2_body,
        grid=(_G,),
        in_specs=[
            pl.BlockSpec((_NC, _B, _D), lambda i: (0, i, 0)),
            pl.BlockSpec((_NC, _B, _D), lambda i: (0, i, 0)),
            pl.BlockSpec((1, _D), lambda i: (0, 0)),
            pl.BlockSpec((_D, _D), lambda i: (0, 0)),
        ],
        out_specs=pl.BlockSpec((_B, _D), lambda i: (i, 0)),
        out_shape=jax.ShapeDtypeStruct((_N, _D), jnp.float32),
    )(p, degp, b1, W2)


def _tc_out(q, degp, b2):
    return pl.pallas_call(
        _tc_out_body,
        grid=(_G,),
        in_specs=[
            pl.BlockSpec((_NC, _B, _D), lambda i: (0, i, 0)),
            pl.BlockSpec((_NC, _B, _D), lambda i: (0, i, 0)),
            pl.BlockSpec((1, _D), lambda i: (0, 0)),
        ],
        out_specs=pl.BlockSpec((_B, _D), lambda i: (i, 0)),
        out_shape=jax.ShapeDtypeStruct((_N, _D), jnp.float32),
    )(q, degp, b2)


def kernel(x, edge_index, W1, b1, W2, b2):
    src = edge_index[0]
    dst = edge_index[1]
    pad_src = jnp.broadcast_to(
        jnp.arange(_PAD, dtype=jnp.int32) % 16, (_NW, _PAD))
    pad_dst = jnp.broadcast_to(
        jnp.arange(_PAD, dtype=jnp.int32) % _NDUM + _N, (_NW, _PAD))
    srcw = jnp.concatenate(
        [src.reshape(_NW, _EW), pad_src], axis=1).reshape(_NW, _K, _CH)
    dstw = jnp.concatenate(
        [dst.reshape(_NW, _EW), pad_dst], axis=1).reshape(_NW, _K, _CH)
    z128 = jnp.zeros((_RZ, _D), jnp.float32)
    ones128 = jnp.ones((_CH, _D), jnp.float32)

    degp = _deg_kernel(dstw, z128, ones128)
    h1 = _tc_lin1(x, W1, degp)
    p1 = _segsum(h1, srcw, dstw, z128)
    h2 = _tc_lin2(p1, degp, b1.reshape(1, _D), W2)
    p2 = _segsum(h2, srcw, dstw, z128)
    return _tc_out(p2, degp, b2.reshape(1, _D))


# speedup vs baseline: 1.2424x; 1.0683x over previous
import functools

import jax
import jax.numpy as jnp
from jax import lax
from jax.experimental import pallas as pl
from jax.experimental.pallas import tpu as pltpu
from jax.experimental.pallas import tpu_sc as plsc

_N = 10000
_D = 128
_E = 320000
_NC = 2
_NS = 16
_NW = _NC * _NS
_EW = _E // _NW
_CH = 128
_K = 80
_NPH = 2
_KP = _K // _NPH
_KP2 = _KP // 2
_PAD = _K * _CH - _EW
_NDUM = 112
_NACC = _N + _NDUM
_RZ = _NACC // _NS

_MESH = dict(core_axis_name="c", subcore_axis_name="s")


@functools.partial(
    pl.kernel,
    mesh=plsc.VectorSubcoreMesh(**_MESH),
    out_type=jax.ShapeDtypeStruct((_NC, _NACC, _D), jnp.float32),
    scratch_types=[
        pltpu.VMEM((_K, _CH), jnp.int32),
        pltpu.VMEM((_CH, _D), jnp.float32),
        pltpu.VMEM_SHARED((_NACC, _D), jnp.float32),
        pltpu.SemaphoreType.DMA,
    ],
)
def _deg_kernel(dst_hbm, z_hbm, ones_hbm, out_hbm, dst_v, ones_v, acc, sem):
    cid = lax.axis_index("c")
    sid = lax.axis_index("s")
    w = cid * _NS + sid
    pltpu.sync_copy(z_hbm, acc.at[pl.ds(sid * _RZ, _RZ)])
    pltpu.sync_copy(dst_hbm.at[w], dst_v)
    pltpu.sync_copy(ones_hbm, ones_v)
    plsc.subcore_barrier()

    def fire(j, carry):
        pltpu.async_copy(ones_v, acc.at[dst_v.at[j]], sem, add=True)
        return carry

    def drain(j, carry):
        pltpu.make_async_copy(ones_v, acc.at[dst_v.at[j]], sem).wait()
        return carry

    lax.fori_loop(0, _K, fire, 0)
    lax.fori_loop(0, _K, drain, 0)
    plsc.subcore_barrier()
    pltpu.sync_copy(acc.at[pl.ds(sid * _RZ, _RZ)],
                    out_hbm.at[cid, pl.ds(sid * _RZ, _RZ)])


@functools.partial(
    pl.kernel,
    mesh=plsc.VectorSubcoreMesh(**_MESH),
    out_type=jax.ShapeDtypeStruct((_NC, _NACC, _D), jnp.float32),
    scratch_types=[
        pltpu.VMEM((_KP, _CH), jnp.int32),
        pltpu.VMEM((_KP, _CH), jnp.int32),
        pltpu.VMEM((_CH, _D), jnp.float32),
        pltpu.VMEM((_CH, _D), jnp.float32),
        pltpu.VMEM_SHARED((_NACC, _D), jnp.float32),
        pltpu.SemaphoreType.DMA,
        pltpu.SemaphoreType.DMA,
    ],
)
def _segsum(h_hbm, src_hbm, dst_hbm, z_hbm, out_hbm,
            src_v, dst_v, rows0, rows1, acc, sem0, sem1):
    cid = lax.axis_index("c")
    sid = lax.axis_index("s")
    w = cid * _NS + sid
    pltpu.sync_copy(z_hbm, acc.at[pl.ds(sid * _RZ, _RZ)])
    plsc.subcore_barrier()

    def pair(i, carry):
        a = 2 * i
        b = a + 1
        pltpu.async_copy(h_hbm.at[src_v.at[b]], rows1, sem1)
        pltpu.make_async_copy(h_hbm.at[src_v.at[a]], rows0, sem0).wait()
        pltpu.sync_copy(rows0, acc.at[dst_v.at[a]], add=True)

        @pl.when(i < _KP2 - 1)
        def _():
            pltpu.async_copy(h_hbm.at[src_v.at[a + 2]], rows0, sem0)

        pltpu.make_async_copy(h_hbm.at[src_v.at[b]], rows1, sem1).wait()
        pltpu.sync_copy(rows1, acc.at[dst_v.at[b]], add=True)
        return carry

    for p in range(_NPH):
        pltpu.sync_copy(src_hbm.at[w, pl.ds(p * _KP, _KP)], src_v)
        pltpu.sync_copy(dst_hbm.at[w, pl.ds(p * _KP, _KP)], dst_v)
        pltpu.async_copy(h_hbm.at[src_v.at[0]], rows0, sem0)
        lax.fori_loop(0, _KP2, pair, 0)
    plsc.subcore_barrier()
    pltpu.sync_copy(acc.at[pl.ds(sid * _RZ, _RZ)],
                    out_hbm.at[cid, pl.ds(sid * _RZ, _RZ)])


_B = 1000
_G = _N // _B


def _dis_block(degp):
    deg = degp[0, :, 0:1] + degp[1, :, 0:1]
    return lax.rsqrt(jnp.maximum(deg, 1.0))


def _tc_lin1_body(x_ref, w_ref, degp_ref, o_ref):
    dis = _dis_block(degp_ref[...])
    o_ref[...] = jnp.dot(x_ref[...], w_ref[...],
                         preferred_element_type=jnp.float32) * dis


def _tc_lin2_body(p_ref, degp_ref, b1_ref, w_ref, o_ref):
    dis = _dis_block(degp_ref[...])
    p = p_ref[...]
    h = jnp.maximum((p[0] + p[1]) * dis + b1_ref[...], 0.0)
    o_ref[...] = jnp.dot(h, w_ref[...],
                         preferred_element_type=jnp.float32) * dis


def _tc_out_body(q_ref, degp_ref, b2_ref, o_ref):
    dis = _dis_block(degp_ref[...])
    q = q_ref[...]
    o_ref[...] = (q[0] + q[1]) * dis + b2_ref[...]


def _tc_lin1(x, W1, degp):
    return pl.pallas_call(
        _tc_lin1_body,
        grid=(_G,),
        in_specs=[
            pl.BlockSpec((_B, _D), lambda i: (i, 0)),
            pl.BlockSpec((_D, _D), lambda i: (0, 0)),
            pl.BlockSpec((_NC, _B, _D), lambda i: (0, i, 0)),
        ],
        out_specs=pl.BlockSpec((_B, _D), lambda i: (i, 0)),
        out_shape=jax.ShapeDtypeStruct((_N, _D), jnp.float32),
    )(x, W1, degp)


def _tc_lin2(p, degp, b1, W2):
    return pl.pallas_call(
        _tc_lin2_body,
        grid=(_G,),
        in_specs=[
            pl.BlockSpec((_NC, _B, _D), lambda i: (0, i, 0)),
            pl.BlockSpec((_NC, _B, _D), lambda i: (0, i, 0)),
            pl.BlockSpec((1, _D), lambda i: (0, 0)),
            pl.BlockSpec((_D, _D), lambda i: (0, 0)),
        ],
        out_specs=pl.BlockSpec((_B, _D), lambda i: (i, 0)),
        out_shape=jax.ShapeDtypeStruct((_N, _D), jnp.float32),
    )(p, degp, b1, W2)


def _tc_out(q, degp, b2):
    return pl.pallas_call(
        _tc_out_body,
        grid=(_G,),
        in_specs=[
            pl.BlockSpec((_NC, _B, _D), lambda i: (0, i, 0)),
            pl.BlockSpec((_NC, _B, _D), lambda i: (0, i, 0)),
            pl.BlockSpec((1, _D), lambda i: (0, 0)),
        ],
        out_specs=pl.BlockSpec((_B, _D), lambda i: (i, 0)),
        out_shape=jax.ShapeDtypeStruct((_N, _D), jnp.float32),
    )(q, degp, b2)


def kernel(x, edge_index, W1, b1, W2, b2):
    src = edge_index[0]
    dst = edge_index[1]
    pad_src = jnp.broadcast_to(
        jnp.arange(_PAD, dtype=jnp.int32) % 16, (_NW, _PAD))
    pad_dst = jnp.broadcast_to(
        jnp.arange(_PAD, dtype=jnp.int32) % _NDUM + _N, (_NW, _PAD))
    srcw = jnp.concatenate(
        [src.reshape(_NW, _EW), pad_src], axis=1).reshape(_NW, _K, _CH)
    dstw = jnp.concatenate(
        [dst.reshape(_NW, _EW), pad_dst], axis=1).reshape(_NW, _K, _CH)
    z128 = jnp.zeros((_RZ, _D), jnp.float32)
    ones128 = jnp.ones((_CH, _D), jnp.float32)

    degp = _deg_kernel(dstw, z128, ones128)
    h1 = _tc_lin1(x, W1, degp)
    p1 = _segsum(h1, srcw, dstw, z128)
    h2 = _tc_lin2(p1, degp, b1.reshape(1, _D), W2)
    p2 = _segsum(h2, srcw, dstw, z128)
    return _tc_out(p2, degp, b2.reshape(1, _D))


# speedup vs baseline: 1.2628x; 1.0164x over previous
import functools

import jax
import jax.numpy as jnp
from jax import lax
from jax.experimental import pallas as pl
from jax.experimental.pallas import tpu as pltpu
from jax.experimental.pallas import tpu_sc as plsc

_N = 10000
_D = 128
_E = 320000
_NC = 2
_NS = 16
_NW = _NC * _NS
_EW = _E // _NW
_CH = 128
_K = 80
_NPH = 2
_KP = _K // _NPH
_KP2 = _KP // 2
_PAD = _K * _CH - _EW
_NDUM = 112
_NACC = _N + _NDUM
_RZ = _NACC // _NS

_MESH = dict(core_axis_name="c", subcore_axis_name="s")


@functools.partial(
    pl.kernel,
    mesh=plsc.VectorSubcoreMesh(**_MESH),
    out_type=jax.ShapeDtypeStruct((_NC, _NACC, _D), jnp.float32),
    scratch_types=[
        pltpu.VMEM((_K, _CH), jnp.int32),
        pltpu.VMEM((_CH, _D), jnp.float32),
        pltpu.VMEM_SHARED((_NACC, _D), jnp.float32),
        pltpu.SemaphoreType.DMA,
    ],
)
def _deg_kernel(dst_hbm, z_hbm, ones_hbm, out_hbm, dst_v, ones_v, acc, sem):
    cid = lax.axis_index("c")
    sid = lax.axis_index("s")
    w = cid * _NS + sid
    pltpu.sync_copy(z_hbm, acc.at[pl.ds(sid * _RZ, _RZ)])
    pltpu.sync_copy(dst_hbm.at[w], dst_v)
    pltpu.sync_copy(ones_hbm, ones_v)
    plsc.subcore_barrier()

    def fire(j, carry):
        pltpu.async_copy(ones_v, acc.at[dst_v.at[j]], sem, add=True)
        return carry

    def drain(j, carry):
        pltpu.make_async_copy(ones_v, acc.at[dst_v.at[j]], sem).wait()
        return carry

    lax.fori_loop(0, _K, fire, 0)
    lax.fori_loop(0, _K, drain, 0)
    plsc.subcore_barrier()
    pltpu.sync_copy(acc.at[pl.ds(sid * _RZ, _RZ)],
                    out_hbm.at[cid, pl.ds(sid * _RZ, _RZ)])


@functools.partial(
    pl.kernel,
    mesh=plsc.VectorSubcoreMesh(**_MESH),
    out_type=jax.ShapeDtypeStruct((_NC, _NACC, _D), jnp.float32),
    scratch_types=[
        pltpu.VMEM((_KP, _CH), jnp.int32),
        pltpu.VMEM((_KP, _CH), jnp.int32),
        pltpu.VMEM((_CH, _D), jnp.float32),
        pltpu.VMEM((_CH, _D), jnp.float32),
        pltpu.VMEM_SHARED((_NACC, _D), jnp.float32),
        pltpu.SemaphoreType.DMA,
        pltpu.SemaphoreType.DMA,
    ],
)
def _segsum(h_hbm, src_hbm, dst_hbm, z_hbm, out_hbm,
            src_v, dst_v, rows0, rows1, acc, sem0, sem1):
    cid = lax.axis_index("c")
    sid = lax.axis_index("s")
    w = cid * _NS + sid
    pltpu.sync_copy(z_hbm, acc.at[pl.ds(sid * _RZ, _RZ)])
    plsc.subcore_barrier()

    def pair(i, carry):
        a = 2 * i
        b = a + 1
        pltpu.async_copy(h_hbm.at[src_v.at[b]], rows1, sem1)
        pltpu.make_async_copy(h_hbm.at[src_v.at[a]], rows0, sem0).wait()
        pltpu.sync_copy(rows0, acc.at[dst_v.at[a]], add=True)

        @pl.when(i < _KP2 - 1)
        def _():
            pltpu.async_copy(h_hbm.at[src_v.at[a + 2]], rows0, sem0)

        pltpu.make_async_copy(h_hbm.at[src_v.at[b]], rows1, sem1).wait()
        pltpu.sync_copy(rows1, acc.at[dst_v.at[b]], add=True)
        return carry

    for p in range(_NPH):
        pltpu.sync_copy(src_hbm.at[w, pl.ds(p * _KP, _KP)], src_v)
        pltpu.sync_copy(dst_hbm.at[w, pl.ds(p * _KP, _KP)], dst_v)
        pltpu.async_copy(h_hbm.at[src_v.at[0]], rows0, sem0)
        lax.fori_loop(0, _KP2, pair, 0)
    plsc.subcore_barrier()
    pltpu.sync_copy(acc.at[pl.ds(sid * _RZ, _RZ)],
                    out_hbm.at[cid, pl.ds(sid * _RZ, _RZ)])


_B = 2000
_G = _N // _B


def _dis_block(degp):
    deg = degp[0, :, 0:1] + degp[1, :, 0:1]
    return lax.rsqrt(jnp.maximum(deg, 1.0))


def _tc_lin1_body(x_ref, w_ref, degp_ref, o_ref):
    dis = _dis_block(degp_ref[...])
    o_ref[...] = jnp.dot(x_ref[...], w_ref[...],
                         preferred_element_type=jnp.float32) * dis


def _tc_lin2_body(p_ref, degp_ref, b1_ref, w_ref, o_ref):
    dis = _dis_block(degp_ref[...])
    p = p_ref[...]
    h = jnp.maximum((p[0] + p[1]) * dis + b1_ref[...], 0.0)
    o_ref[...] = jnp.dot(h, w_ref[...],
                         preferred_element_type=jnp.float32) * dis


def _tc_out_body(q_ref, degp_ref, b2_ref, o_ref):
    dis = _dis_block(degp_ref[...])
    q = q_ref[...]
    o_ref[...] = (q[0] + q[1]) * dis + b2_ref[...]


def _tc_lin1(x, W1, degp):
    return pl.pallas_call(
        _tc_lin1_body,
        grid=(_G,),
        in_specs=[
            pl.BlockSpec((_B, _D), lambda i: (i, 0)),
            pl.BlockSpec((_D, _D), lambda i: (0, 0)),
            pl.BlockSpec((_NC, _B, _D), lambda i: (0, i, 0)),
        ],
        out_specs=pl.BlockSpec((_B, _D), lambda i: (i, 0)),
        out_shape=jax.ShapeDtypeStruct((_N, _D), jnp.float32),
    )(x, W1, degp)


def _tc_lin2(p, degp, b1, W2):
    return pl.pallas_call(
        _tc_lin2_body,
        grid=(_G,),
        in_specs=[
            pl.BlockSpec((_NC, _B, _D), lambda i: (0, i, 0)),
            pl.BlockSpec((_NC, _B, _D), lambda i: (0, i, 0)),
            pl.BlockSpec((1, _D), lambda i: (0, 0)),
            pl.BlockSpec((_D, _D), lambda i: (0, 0)),
        ],
        out_specs=pl.BlockSpec((_B, _D), lambda i: (i, 0)),
        out_shape=jax.ShapeDtypeStruct((_N, _D), jnp.float32),
    )(p, degp, b1, W2)


def _tc_out(q, degp, b2):
    return pl.pallas_call(
        _tc_out_body,
        grid=(_G,),
        in_specs=[
            pl.BlockSpec((_NC, _B, _D), lambda i: (0, i, 0)),
            pl.BlockSpec((_NC, _B, _D), lambda i: (0, i, 0)),
            pl.BlockSpec((1, _D), lambda i: (0, 0)),
        ],
        out_specs=pl.BlockSpec((_B, _D), lambda i: (i, 0)),
        out_shape=jax.ShapeDtypeStruct((_N, _D), jnp.float32),
    )(q, degp, b2)


def kernel(x, edge_index, W1, b1, W2, b2):
    src = edge_index[0]
    dst = edge_index[1]
    pad_src = jnp.broadcast_to(
        jnp.arange(_PAD, dtype=jnp.int32) % 16, (_NW, _PAD))
    pad_dst = jnp.broadcast_to(
        jnp.arange(_PAD, dtype=jnp.int32) % _NDUM + _N, (_NW, _PAD))
    srcw = jnp.concatenate(
        [src.reshape(_NW, _EW), pad_src], axis=1).reshape(_NW, _K, _CH)
    dstw = jnp.concatenate(
        [dst.reshape(_NW, _EW), pad_dst], axis=1).reshape(_NW, _K, _CH)
    z128 = jnp.zeros((_RZ, _D), jnp.float32)
    ones128 = jnp.ones((_CH, _D), jnp.float32)

    degp = _deg_kernel(dstw, z128, ones128)
    h1 = _tc_lin1(x, W1, degp)
    p1 = _segsum(h1, srcw, dstw, z128)
    h2 = _tc_lin2(p1, degp, b1.reshape(1, _D), W2)
    p2 = _segsum(h2, srcw, dstw, z128)
    return _tc_out(p2, degp, b2.reshape(1, _D))


# speedup vs baseline: 1.2720x; 1.0073x over previous
import functools

import jax
import jax.numpy as jnp
from jax import lax
from jax.experimental import pallas as pl
from jax.experimental.pallas import tpu as pltpu
from jax.experimental.pallas import tpu_sc as plsc

_N = 10000
_D = 128
_E = 320000
_NC = 2
_NS = 16
_NW = _NC * _NS
_EW = _E // _NW
_CH = 128
_K = 80
_NPH = 2
_KP = _K // _NPH
_KP2 = _KP // 2
_PAD = _K * _CH - _EW
_NDUM = 112
_NACC = _N + _NDUM
_RZ = _NACC // _NS

_MESH = dict(core_axis_name="c", subcore_axis_name="s")


@functools.partial(
    pl.kernel,
    mesh=plsc.VectorSubcoreMesh(**_MESH),
    out_type=jax.ShapeDtypeStruct((_NC, _NACC, _D), jnp.float32),
    scratch_types=[
        pltpu.VMEM((_K, _CH), jnp.int32),
        pltpu.VMEM((_CH, _D), jnp.float32),
        pltpu.VMEM_SHARED((_NACC, _D), jnp.float32),
        pltpu.SemaphoreType.DMA,
    ],
)
def _deg_kernel(dst_hbm, z_hbm, ones_hbm, out_hbm, dst_v, ones_v, acc, sem):
    cid = lax.axis_index("c")
    sid = lax.axis_index("s")
    w = cid * _NS + sid
    pltpu.sync_copy(z_hbm, acc.at[pl.ds(sid * _RZ, _RZ)])
    pltpu.sync_copy(dst_hbm.at[w], dst_v)
    pltpu.sync_copy(ones_hbm, ones_v)
    plsc.subcore_barrier()

    def fire(j, carry):
        pltpu.async_copy(ones_v, acc.at[dst_v.at[j]], sem, add=True)
        return carry

    def drain(j, carry):
        pltpu.make_async_copy(ones_v, acc.at[dst_v.at[j]], sem).wait()
        return carry

    lax.fori_loop(0, _K, fire, 0)
    lax.fori_loop(0, _K, drain, 0)
    plsc.subcore_barrier()
    pltpu.sync_copy(acc.at[pl.ds(sid * _RZ, _RZ)],
                    out_hbm.at[cid, pl.ds(sid * _RZ, _RZ)])


@functools.partial(
    pl.kernel,
    mesh=plsc.VectorSubcoreMesh(**_MESH),
    out_type=jax.ShapeDtypeStruct((_NC, _NACC, _D), jnp.float32),
    scratch_types=[
        pltpu.VMEM((_KP, _CH), jnp.int32),
        pltpu.VMEM((_KP, _CH), jnp.int32),
        pltpu.VMEM((_CH, _D), jnp.float32),
        pltpu.VMEM((_CH, _D), jnp.float32),
        pltpu.VMEM_SHARED((_NACC, _D), jnp.float32),
        pltpu.SemaphoreType.DMA,
        pltpu.SemaphoreType.DMA,
    ],
)
def _segsum(h_hbm, src_hbm, dst_hbm, z_hbm, out_hbm,
            src_v, dst_v, rows0, rows1, acc, sem0, sem1):
    cid = lax.axis_index("c")
    sid = lax.axis_index("s")
    w = cid * _NS + sid
    pltpu.sync_copy(z_hbm, acc.at[pl.ds(sid * _RZ, _RZ)])
    plsc.subcore_barrier()

    def pair(i, carry):
        a = 2 * i
        b = a + 1
        pltpu.async_copy(h_hbm.at[src_v.at[b]], rows1, sem1)
        pltpu.make_async_copy(h_hbm.at[src_v.at[a]], rows0, sem0).wait()
        pltpu.sync_copy(rows0, acc.at[dst_v.at[a]], add=True)

        @pl.when(i < _KP2 - 1)
        def _():
            pltpu.async_copy(h_hbm.at[src_v.at[a + 2]], rows0, sem0)

        pltpu.make_async_copy(h_hbm.at[src_v.at[b]], rows1, sem1).wait()
        pltpu.sync_copy(rows1, acc.at[dst_v.at[b]], add=True)
        return carry

    for p in range(_NPH):
        pltpu.sync_copy(src_hbm.at[w, pl.ds(p * _KP, _KP)], src_v)
        pltpu.sync_copy(dst_hbm.at[w, pl.ds(p * _KP, _KP)], dst_v)
        pltpu.async_copy(h_hbm.at[src_v.at[0]], rows0, sem0)
        lax.fori_loop(0, _KP2, pair, 0)
    plsc.subcore_barrier()
    pltpu.sync_copy(acc.at[pl.ds(sid * _RZ, _RZ)],
                    out_hbm.at[cid, pl.ds(sid * _RZ, _RZ)])


_B = 5000
_G = _N // _B


def _dis_block(degp):
    deg = degp[0, :, 0:1] + degp[1, :, 0:1]
    return lax.rsqrt(jnp.maximum(deg, 1.0))


def _tc_lin1_body(x_ref, w_ref, degp_ref, o_ref):
    dis = _dis_block(degp_ref[...])
    o_ref[...] = jnp.dot(x_ref[...], w_ref[...],
                         preferred_element_type=jnp.float32) * dis


def _tc_lin2_body(p_ref, degp_ref, b1_ref, w_ref, o_ref):
    dis = _dis_block(degp_ref[...])
    p = p_ref[...]
    h = jnp.maximum((p[0] + p[1]) * dis + b1_ref[...], 0.0)
    o_ref[...] = jnp.dot(h, w_ref[...],
                         preferred_element_type=jnp.float32) * dis


def _tc_out_body(q_ref, degp_ref, b2_ref, o_ref):
    dis = _dis_block(degp_ref[...])
    q = q_ref[...]
    o_ref[...] = (q[0] + q[1]) * dis + b2_ref[...]


def _tc_lin1(x, W1, degp):
    return pl.pallas_call(
        _tc_lin1_body,
        grid=(_G,),
        in_specs=[
            pl.BlockSpec((_B, _D), lambda i: (i, 0)),
            pl.BlockSpec((_D, _D), lambda i: (0, 0)),
            pl.BlockSpec((_NC, _B, _D), lambda i: (0, i, 0)),
        ],
        out_specs=pl.BlockSpec((_B, _D), lambda i: (i, 0)),
        out_shape=jax.ShapeDtypeStruct((_N, _D), jnp.float32),
    )(x, W1, degp)


def _tc_lin2(p, degp, b1, W2):
    return pl.pallas_call(
        _tc_lin2_body,
        grid=(_G,),
        in_specs=[
            pl.BlockSpec((_NC, _B, _D), lambda i: (0, i, 0)),
            pl.BlockSpec((_NC, _B, _D), lambda i: (0, i, 0)),
            pl.BlockSpec((1, _D), lambda i: (0, 0)),
            pl.BlockSpec((_D, _D), lambda i: (0, 0)),
        ],
        out_specs=pl.BlockSpec((_B, _D), lambda i: (i, 0)),
        out_shape=jax.ShapeDtypeStruct((_N, _D), jnp.float32),
    )(p, degp, b1, W2)


def _tc_out(q, degp, b2):
    return pl.pallas_call(
        _tc_out_body,
        grid=(_G,),
        in_specs=[
            pl.BlockSpec((_NC, _B, _D), lambda i: (0, i, 0)),
            pl.BlockSpec((_NC, _B, _D), lambda i: (0, i, 0)),
            pl.BlockSpec((1, _D), lambda i: (0, 0)),
        ],
        out_specs=pl.BlockSpec((_B, _D), lambda i: (i, 0)),
        out_shape=jax.ShapeDtypeStruct((_N, _D), jnp.float32),
    )(q, degp, b2)


def kernel(x, edge_index, W1, b1, W2, b2):
    src = edge_index[0]
    dst = edge_index[1]
    pad_src = jnp.broadcast_to(
        jnp.arange(_PAD, dtype=jnp.int32) % 16, (_NW, _PAD))
    pad_dst = jnp.broadcast_to(
        jnp.arange(_PAD, dtype=jnp.int32) % _NDUM + _N, (_NW, _PAD))
    srcw = jnp.concatenate(
        [src.reshape(_NW, _EW), pad_src], axis=1).reshape(_NW, _K, _CH)
    dstw = jnp.concatenate(
        [dst.reshape(_NW, _EW), pad_dst], axis=1).reshape(_NW, _K, _CH)
    z128 = jnp.zeros((_RZ, _D), jnp.float32)
    ones128 = jnp.ones((_CH, _D), jnp.float32)

    degp = _deg_kernel(dstw, z128, ones128)
    h1 = _tc_lin1(x, W1, degp)
    p1 = _segsum(h1, srcw, dstw, z128)
    h2 = _tc_lin2(p1, degp, b1.reshape(1, _D), W2)
    p2 = _segsum(h2, srcw, dstw, z128)
    return _tc_out(p2, degp, b2.reshape(1, _D))
